# baseline (speedup 1.0000x reference)
import functools
import math

import jax
import jax.numpy as jnp
from jax import lax
from jax.experimental import pallas as pl
from jax.experimental.pallas import tpu as pltpu
from jax.experimental.pallas import tpu_sc as plsc

_BH = 4
_WT = 32
_R = 16


def _tc_body(pt_ref, es_ref, pid_ref, tid_ref, num_ref, temp_ref,
             w_ref, wn_ref, wt_ref, bias_ref, g_ref, b_ref, pe_ref, o_ref):
    M = pt_ref.shape[0]
    S = pe_ref.shape[0]
    R = M // S
    P = pid_ref.shape[2]
    D = pt_ref.shape[1] // 2
    ptv = pt_ref[...].astype(jnp.float32)
    esv = es_ref[...].astype(jnp.float32)
    pidf = (pid_ref[...].reshape(M, P) != 0).astype(jnp.float32)
    tidf = (tid_ref[...].reshape(M, P) != 0).astype(jnp.float32)
    rp = 1.0 / jnp.maximum(jnp.sum(pidf, axis=1, keepdims=True), 1.0)
    rt = 1.0 / jnp.maximum(jnp.sum(tidf, axis=1, keepdims=True), 1.0)
    lane = lax.broadcasted_iota(jnp.int32, (1, 2 * D), 1)
    scale = jnp.where(lane < D, rp, rt)
    cat = jnp.concatenate([ptv * scale, esv[:, :D]], axis=1)
    x = jnp.dot(cat, w_ref[...], preferred_element_type=jnp.float32)
    x = x + jnp.dot(num_ref[...].reshape(M, num_ref.shape[2]), wn_ref[...],
                    preferred_element_type=jnp.float32)
    x = x + jnp.dot(temp_ref[...].reshape(M, temp_ref.shape[2]), wt_ref[...],
                    preferred_element_type=jnp.float32)
    x = x + bias_ref[...]
    mu = jnp.mean(x, axis=1, keepdims=True)
    xc = x - mu
    var = jnp.mean(xc * xc, axis=1, keepdims=True)
    y = xc * lax.rsqrt(var + 1e-5) * g_ref[...] + b_ref[...]
    y = 0.5 * y * (1.0 + lax.erf(y * (1.0 / math.sqrt(2.0))))
    o_ref[...] = y.reshape(R, S, y.shape[1]) + pe_ref[...][None]


def kernel(event_type_ids, proc_path_ids, tgt_path_ids, signing_ids,
           numerical, temporal, event_table, proc_path_table, tgt_path_table,
           signing_table, num_W, num_b, temp_W, temp_b, proj_W, proj_b,
           ln_gamma, ln_beta, pe):
    B, S, P = proc_path_ids.shape
    BT = B * S
    E = event_table.shape[1]
    D = proc_path_table.shape[1]
    SG = signing_table.shape[1]
    DM = proj_W.shape[1]
    f32 = jnp.float32

    info = plsc.get_sparse_core_info()
    NC, NS = info.num_cores, info.num_subcores
    NW = NC * NS
    bw = B // NW
    BH = _BH
    nhg = bw // BH
    HT = BH * S
    WT = _WT
    NWAVE = HT // WT
    NIR = HT * P // 128
    NPAIR = NWAVE // 2

    mesh = plsc.VectorSubcoreMesh(core_axis_name="c", subcore_axis_name="s")

    @functools.partial(
        pl.kernel,
        out_type=(jax.ShapeDtypeStruct((BT, 2 * D), jnp.bfloat16),
                  jax.ShapeDtypeStruct((BT, 2 * D), jnp.bfloat16)),
        mesh=mesh,
        scratch_types=[
            pltpu.VMEM((BH, S, P), jnp.int32),
            pltpu.VMEM((BH, S, P), jnp.int32),
            pltpu.VMEM((NIR, 128), jnp.int32),
            pltpu.VMEM((NIR, 128), jnp.int32),
            pltpu.VMEM((2, WT * P, D), jnp.bfloat16),
            pltpu.VMEM((2, WT * P, D), jnp.bfloat16),
            pltpu.VMEM((2, WT, 2 * D), jnp.bfloat16),
            pltpu.VMEM((BH, S), jnp.int32),
            pltpu.VMEM((BH, S), jnp.int32),
            pltpu.VMEM((2, S, E), jnp.bfloat16),
            pltpu.VMEM((2, S, SG), jnp.bfloat16),
            pltpu.SemaphoreType.DMA,
            pltpu.SemaphoreType.DMA,
            pltpu.SemaphoreType.DMA,
            pltpu.SemaphoreType.DMA,
            pltpu.SemaphoreType.DMA,
        ],
        compiler_params=pltpu.CompilerParams(use_tc_tiling_on_sc=False,
                                             needs_layout_passes=False),
    )
    def sc_pool(pids_h, tids_h, eids_h, sids_h, ptab_h, ttab_h, etab_h,
                stab_h, pt_out, es_out,
                pstg, tstg, pidx, tidx, pbuf, tbuf, ptb, eidg, sidg,
                erow, srow, sem_g0, sem_g1, sem_o, sem_es, sem_eso):
        wid = lax.axis_index("s") * NC + lax.axis_index("c")
        b_base = wid * bw
        lanes = lax.iota(jnp.int32, 16)
        rb = lanes >> 3
        cb = lanes & 7
        gsem = (sem_g0, sem_g1)

        def fire_wave(w, k):
            for u in range(2):
                hsl = pl.ds(u * 128, 128)
                pltpu.async_copy(ptab_h.at[pidx.at[2 * w + u]],
                                 pbuf.at[k, hsl], gsem[k])
                pltpu.async_copy(ttab_h.at[tidx.at[2 * w + u]],
                                 tbuf.at[k, hsl], gsem[k])

        def wait_wave(k):
            for u in range(2):
                hsl = pl.ds(u * 128, 128)
                pltpu.make_async_copy(ptab_h.at[pidx.at[0]],
                                      pbuf.at[k, hsl], gsem[k]).wait()
                pltpu.make_async_copy(ttab_h.at[tidx.at[0]],
                                      tbuf.at[k, hsl], gsem[k]).wait()

        def compute_wave(k, hgtok, w):
            @pl.loop(0, WT)
            def _tok(t):
                base = t * P

                def tree8(buf, sl):
                    r = [buf[k, base + q, sl] for q in range(P)]
                    while len(r) > 1:
                        r = [r[2 * q] + r[2 * q + 1]
                             for q in range(len(r) // 2)]
                    return r[0]

                for c2 in range(D // 32):
                    sl = pl.ds(c2 * 32, 32)
                    ptb[k, t, sl] = tree8(pbuf, sl)
                    ptb[k, t, pl.ds(D + c2 * 32, 32)] = tree8(tbuf, sl)

            pltpu.async_copy(
                ptb.at[k], pt_out.at[pl.ds(hgtok + w * WT, WT), :], sem_o)

        def drain_out():
            pltpu.make_async_copy(
                ptb.at[0], pt_out.at[pl.ds(0, WT), :], sem_o).wait()

        @pl.loop(0, nhg)
        def _hg(h):
            b0 = b_base + h * BH
            hgtok = pl.multiple_of(b0 * S, 8)
            pltpu.sync_copy(pids_h.at[pl.ds(b0, BH), :, :], pstg)
            pltpu.sync_copy(tids_h.at[pl.ds(b0, BH), :, :], tstg)
            kk_b = S * P // 16

            @pl.loop(0, HT * P // 16)
            def _rp(kk):
                b_ = kk // kk_b
                bv = lanes * 0 + b_
                sv = (kk % kk_b) * 2 + rb
                r_ = kk >> 3
                c_ = (kk & 7) * 16
                pidx[r_, pl.ds(c_, 16)] = plsc.load_gather(
                    pstg, [bv, sv, cb])
                tidx[r_, pl.ds(c_, 16)] = plsc.load_gather(
                    tstg, [bv, sv, cb])

            fire_wave(0, 0)

            pltpu.sync_copy(eids_h.at[pl.ds(b0, BH), :], eidg)
            pltpu.sync_copy(sids_h.at[pl.ds(b0, BH), :], sidg)
            for r in range(BH):
                k = r % 2
                if r >= 2:
                    pltpu.make_async_copy(
                        erow.at[k], es_out.at[pl.ds(0, S), pl.ds(0, E)],
                        sem_eso).wait()
                    pltpu.make_async_copy(
                        srow.at[k], es_out.at[pl.ds(0, S), pl.ds(E, SG)],
                        sem_eso).wait()
                cps = []
                for (lo, ln) in ((0, 128), (128, S - 128)):
                    cps.append(pltpu.async_copy(
                        etab_h.at[eidg.at[r, pl.ds(lo, ln)]],
                        erow.at[k, pl.ds(lo, ln)], sem_es))
                    cps.append(pltpu.async_copy(
                        stab_h.at[sidg.at[r, pl.ds(lo, ln)]],
                        srow.at[k, pl.ds(lo, ln)], sem_es))
                for cp in cps:
                    cp.wait()
                rtok = pl.multiple_of(hgtok + r * S, 8)
                pltpu.async_copy(
                    erow.at[k], es_out.at[pl.ds(rtok, S), pl.ds(0, E)],
                    sem_eso)
                pltpu.async_copy(
                    srow.at[k], es_out.at[pl.ds(rtok, S), pl.ds(E, SG)],
                    sem_eso)

            @pl.loop(0, NPAIR)
            def _pair(j):
                w0 = 2 * j
                fire_wave(w0 + 1, 1)
                wait_wave(0)

                @pl.when(j > 0)
                def _lazy():
                    drain_out()
                    drain_out()

                compute_wave(0, hgtok, w0)
                fire_wave(w0 + 2, 0)
                wait_wave(1)
                compute_wave(1, hgtok, w0 + 1)

            wait_wave(0)
            drain_out()
            drain_out()
            compute_wave(0, hgtok, NWAVE - 1)
            drain_out()
            for _ in range(4):
                pltpu.make_async_copy(
                    erow.at[0], es_out.at[pl.ds(0, S), pl.ds(0, E)],
                    sem_eso).wait()

    pt_sum, es_emb = sc_pool(
        proc_path_ids, tgt_path_ids, event_type_ids, signing_ids,
        proc_path_table.astype(jnp.bfloat16),
        tgt_path_table.astype(jnp.bfloat16),
        event_table.astype(jnp.bfloat16),
        signing_table.astype(jnp.bfloat16))

    NW_ = proj_W[E + 2 * D + SG:E + 2 * D + SG + num_W.shape[1]]
    TW_ = proj_W[E + 2 * D + SG + num_W.shape[1]:]
    wn = num_W @ NW_
    wt = temp_W @ TW_
    bias2 = (proj_b + num_b @ NW_ + temp_b @ TW_)[None, :]
    wcat = jnp.concatenate([proj_W[E:E + D], proj_W[E + D:E + 2 * D],
                            proj_W[:E], proj_W[E + 2 * D:E + 2 * D + SG]],
                           axis=0)
    pe_s = pe[0, :S, :]

    R = _R
    M = R * S
    out = pl.pallas_call(
        _tc_body,
        grid=(B // R,),
        in_specs=[
            pl.BlockSpec((M, 2 * D), lambda i: (i, 0)),
            pl.BlockSpec((M, 2 * D), lambda i: (i, 0)),
            pl.BlockSpec((R, S, P), lambda i: (i, 0, 0)),
            pl.BlockSpec((R, S, P), lambda i: (i, 0, 0)),
            pl.BlockSpec((R, S, numerical.shape[2]), lambda i: (i, 0, 0)),
            pl.BlockSpec((R, S, temporal.shape[2]), lambda i: (i, 0, 0)),
            pl.BlockSpec((E + 2 * D + SG, DM), lambda i: (0, 0)),
            pl.BlockSpec((numerical.shape[2], DM), lambda i: (0, 0)),
            pl.BlockSpec((temporal.shape[2], DM), lambda i: (0, 0)),
            pl.BlockSpec((1, DM), lambda i: (0, 0)),
            pl.BlockSpec((1, DM), lambda i: (0, 0)),
            pl.BlockSpec((1, DM), lambda i: (0, 0)),
            pl.BlockSpec((S, DM), lambda i: (0, 0)),
        ],
        out_specs=pl.BlockSpec((R, S, DM), lambda i: (i, 0, 0)),
        out_shape=jax.ShapeDtypeStruct((B, S, DM), f32),
    )(pt_sum, es_emb, proc_path_ids, tgt_path_ids,
      numerical, temporal, wcat, wn, wt, bias2,
      ln_gamma[None, :], ln_beta[None, :], pe_s)
    return out


# speedup vs baseline: 1.2928x; 1.2928x over previous
import functools
import math

import jax
import jax.numpy as jnp
from jax import lax
from jax.experimental import pallas as pl
from jax.experimental.pallas import tpu as pltpu
from jax.experimental.pallas import tpu_sc as plsc

_BH = 4
_WT = 32
_R = 16


def _tc_body(pt_ref, es_ref, pid_ref, tid_ref, num_ref, temp_ref,
             w_ref, wn_ref, wt_ref, bias_ref, g_ref, b_ref, pe_ref, o_ref):
    M = pt_ref.shape[0]
    S = pe_ref.shape[0]
    R = M // S
    P = pid_ref.shape[2]
    D = pt_ref.shape[1] // 2
    ptv = pt_ref[...].astype(jnp.float32)
    esv = es_ref[...].astype(jnp.float32)
    pidf = (pid_ref[...].reshape(M, P) != 0).astype(jnp.float32)
    tidf = (tid_ref[...].reshape(M, P) != 0).astype(jnp.float32)
    rp = 1.0 / jnp.maximum(jnp.sum(pidf, axis=1, keepdims=True), 1.0)
    rt = 1.0 / jnp.maximum(jnp.sum(tidf, axis=1, keepdims=True), 1.0)
    lane = lax.broadcasted_iota(jnp.int32, (1, 2 * D), 1)
    scale = jnp.where(lane < D, rp, rt)
    cat = jnp.concatenate([ptv * scale, esv[:, :D]], axis=1)
    x = jnp.dot(cat, w_ref[...], preferred_element_type=jnp.float32)
    x = x + jnp.dot(num_ref[...].reshape(M, num_ref.shape[2]), wn_ref[...],
                    preferred_element_type=jnp.float32)
    x = x + jnp.dot(temp_ref[...].reshape(M, temp_ref.shape[2]), wt_ref[...],
                    preferred_element_type=jnp.float32)
    x = x + bias_ref[...]
    mu = jnp.mean(x, axis=1, keepdims=True)
    xc = x - mu
    var = jnp.mean(xc * xc, axis=1, keepdims=True)
    y = xc * lax.rsqrt(var + 1e-5) * g_ref[...] + b_ref[...]
    y = 0.5 * y * (1.0 + lax.erf(y * (1.0 / math.sqrt(2.0))))
    o_ref[...] = y.reshape(R, S, y.shape[1]) + pe_ref[...][None]


def kernel(event_type_ids, proc_path_ids, tgt_path_ids, signing_ids,
           numerical, temporal, event_table, proc_path_table, tgt_path_table,
           signing_table, num_W, num_b, temp_W, temp_b, proj_W, proj_b,
           ln_gamma, ln_beta, pe):
    B, S, P = proc_path_ids.shape
    BT = B * S
    E = event_table.shape[1]
    D = proc_path_table.shape[1]
    SG = signing_table.shape[1]
    DM = proj_W.shape[1]
    f32 = jnp.float32

    info = plsc.get_sparse_core_info()
    NC, NS = info.num_cores, info.num_subcores
    NW = NC * NS
    bw = B // NW
    BH = _BH
    nhg = bw // BH
    HT = BH * S
    WT = _WT
    NWAVE = HT // WT
    NIR = HT * P // 128
    NPAIR = NWAVE // 2

    mesh = plsc.VectorSubcoreMesh(core_axis_name="c", subcore_axis_name="s")

    @functools.partial(
        pl.kernel,
        out_type=(jax.ShapeDtypeStruct((BT, 2 * D), f32),
                  jax.ShapeDtypeStruct((BT, 2 * D), f32)),
        mesh=mesh,
        scratch_types=[
            pltpu.VMEM((BH, S, P), jnp.int32),
            pltpu.VMEM((BH, S, P), jnp.int32),
            pltpu.VMEM((NIR, 128), jnp.int32),
            pltpu.VMEM((NIR, 128), jnp.int32),
            pltpu.VMEM((2, WT * P, D), jnp.bfloat16),
            pltpu.VMEM((2, WT * P, D), jnp.bfloat16),
            pltpu.VMEM((2, WT, 2 * D), f32),
            pltpu.VMEM((BH, S), jnp.int32),
            pltpu.VMEM((BH, S), jnp.int32),
            pltpu.VMEM((2, S, E), f32),
            pltpu.VMEM((2, S, SG), f32),
            pltpu.SemaphoreType.DMA,
            pltpu.SemaphoreType.DMA,
            pltpu.SemaphoreType.DMA,
            pltpu.SemaphoreType.DMA,
            pltpu.SemaphoreType.DMA,
        ],
        compiler_params=pltpu.CompilerParams(use_tc_tiling_on_sc=False,
                                             needs_layout_passes=False),
    )
    def sc_pool(pids_h, tids_h, eids_h, sids_h, ptab_h, ttab_h, etab_h,
                stab_h, pt_out, es_out,
                pstg, tstg, pidx, tidx, pbuf, tbuf, ptb, eidg, sidg,
                erow, srow, sem_g0, sem_g1, sem_o, sem_es, sem_eso):
        wid = lax.axis_index("s") * NC + lax.axis_index("c")
        b_base = wid * bw
        lanes = lax.iota(jnp.int32, 16)
        rb = lanes >> 3
        cb = lanes & 7
        gsem = (sem_g0, sem_g1)

        def fire_wave(w, k):
            for u in range(2):
                hsl = pl.ds(u * 128, 128)
                pltpu.async_copy(ptab_h.at[pidx.at[2 * w + u]],
                                 pbuf.at[k, hsl], gsem[k])
                pltpu.async_copy(ttab_h.at[tidx.at[2 * w + u]],
                                 tbuf.at[k, hsl], gsem[k])

        def wait_wave(k):
            for u in range(2):
                hsl = pl.ds(u * 128, 128)
                pltpu.make_async_copy(ptab_h.at[pidx.at[0]],
                                      pbuf.at[k, hsl], gsem[k]).wait()
                pltpu.make_async_copy(ttab_h.at[tidx.at[0]],
                                      tbuf.at[k, hsl], gsem[k]).wait()

        def compute_wave(k, hgtok, w):
            @pl.loop(0, WT)
            def _tok(t):
                base = t * P

                def tree8(buf, sl):
                    r = [buf[k, base + q, sl] for q in range(P)]
                    while len(r) > 1:
                        r = [r[2 * q] + r[2 * q + 1]
                             for q in range(len(r) // 2)]
                    return plsc.unpack(r[0], format=plsc.PackFormat.INTERLEAVED)

                for c2 in range(D // 32):
                    sl = pl.ds(c2 * 32, 32)
                    pa, pb = tree8(pbuf, sl)
                    ptb[k, t, pl.ds(c2 * 32, 16)] = pa
                    ptb[k, t, pl.ds(c2 * 32 + 16, 16)] = pb
                    ta, tb = tree8(tbuf, sl)
                    ptb[k, t, pl.ds(D + c2 * 32, 16)] = ta
                    ptb[k, t, pl.ds(D + c2 * 32 + 16, 16)] = tb

            pltpu.async_copy(
                ptb.at[k], pt_out.at[pl.ds(hgtok + w * WT, WT), :], sem_o)

        def drain_out():
            pltpu.make_async_copy(
                ptb.at[0], pt_out.at[pl.ds(0, WT), :], sem_o).wait()

        @pl.loop(0, nhg)
        def _hg(h):
            b0 = b_base + h * BH
            hgtok = pl.multiple_of(b0 * S, 8)
            pltpu.sync_copy(pids_h.at[pl.ds(b0, BH), :, :], pstg)
            pltpu.sync_copy(tids_h.at[pl.ds(b0, BH), :, :], tstg)
            kk_b = S * P // 16

            @pl.loop(0, HT * P // 16)
            def _rp(kk):
                b_ = kk // kk_b
                bv = lanes * 0 + b_
                sv = (kk % kk_b) * 2 + rb
                r_ = kk >> 3
                c_ = (kk & 7) * 16
                pidx[r_, pl.ds(c_, 16)] = plsc.load_gather(
                    pstg, [bv, sv, cb])
                tidx[r_, pl.ds(c_, 16)] = plsc.load_gather(
                    tstg, [bv, sv, cb])

            fire_wave(0, 0)

            pltpu.sync_copy(eids_h.at[pl.ds(b0, BH), :], eidg)
            pltpu.sync_copy(sids_h.at[pl.ds(b0, BH), :], sidg)
            for r in range(BH):
                k = r % 2
                if r >= 2:
                    pltpu.make_async_copy(
                        erow.at[k], es_out.at[pl.ds(0, S), pl.ds(0, E)],
                        sem_eso).wait()
                    pltpu.make_async_copy(
                        srow.at[k], es_out.at[pl.ds(0, S), pl.ds(E, SG)],
                        sem_eso).wait()
                cps = []
                for (lo, ln) in ((0, 128), (128, S - 128)):
                    cps.append(pltpu.async_copy(
                        etab_h.at[eidg.at[r, pl.ds(lo, ln)]],
                        erow.at[k, pl.ds(lo, ln)], sem_es))
                    cps.append(pltpu.async_copy(
                        stab_h.at[sidg.at[r, pl.ds(lo, ln)]],
                        srow.at[k, pl.ds(lo, ln)], sem_es))
                for cp in cps:
                    cp.wait()
                rtok = pl.multiple_of(hgtok + r * S, 8)
                pltpu.async_copy(
                    erow.at[k], es_out.at[pl.ds(rtok, S), pl.ds(0, E)],
                    sem_eso)
                pltpu.async_copy(
                    srow.at[k], es_out.at[pl.ds(rtok, S), pl.ds(E, SG)],
                    sem_eso)

            @pl.loop(0, NPAIR)
            def _pair(j):
                w0 = 2 * j
                fire_wave(w0 + 1, 1)
                wait_wave(0)

                @pl.when(j > 0)
                def _lazy():
                    drain_out()
                    drain_out()

                compute_wave(0, hgtok, w0)
                fire_wave(w0 + 2, 0)
                wait_wave(1)
                compute_wave(1, hgtok, w0 + 1)

            wait_wave(0)
            drain_out()
            drain_out()
            compute_wave(0, hgtok, NWAVE - 1)
            drain_out()
            for _ in range(4):
                pltpu.make_async_copy(
                    erow.at[0], es_out.at[pl.ds(0, S), pl.ds(0, E)],
                    sem_eso).wait()

    pt_sum, es_emb = sc_pool(
        proc_path_ids, tgt_path_ids, event_type_ids, signing_ids,
        proc_path_table.astype(jnp.bfloat16),
        tgt_path_table.astype(jnp.bfloat16),
        event_table, signing_table)

    NW_ = proj_W[E + 2 * D + SG:E + 2 * D + SG + num_W.shape[1]]
    TW_ = proj_W[E + 2 * D + SG + num_W.shape[1]:]
    wn = num_W @ NW_
    wt = temp_W @ TW_
    bias2 = (proj_b + num_b @ NW_ + temp_b @ TW_)[None, :]
    perm = jnp.concatenate(
        [jnp.arange(0, 32, 2, dtype=jnp.int32) + 32 * blk2 + off
         for blk2 in range(D // 32) for off in (0, 1)])
    wp = proj_W[E:E + D][perm]
    wtg = proj_W[E + D:E + 2 * D][perm]
    wcat = jnp.concatenate([wp, wtg, proj_W[:E],
                            proj_W[E + 2 * D:E + 2 * D + SG]],
                           axis=0)
    pe_s = pe[0, :S, :]

    R = _R
    M = R * S
    out = pl.pallas_call(
        _tc_body,
        grid=(B // R,),
        in_specs=[
            pl.BlockSpec((M, 2 * D), lambda i: (i, 0)),
            pl.BlockSpec((M, 2 * D), lambda i: (i, 0)),
            pl.BlockSpec((R, S, P), lambda i: (i, 0, 0)),
            pl.BlockSpec((R, S, P), lambda i: (i, 0, 0)),
            pl.BlockSpec((R, S, numerical.shape[2]), lambda i: (i, 0, 0)),
            pl.BlockSpec((R, S, temporal.shape[2]), lambda i: (i, 0, 0)),
            pl.BlockSpec((E + 2 * D + SG, DM), lambda i: (0, 0)),
            pl.BlockSpec((numerical.shape[2], DM), lambda i: (0, 0)),
            pl.BlockSpec((temporal.shape[2], DM), lambda i: (0, 0)),
            pl.BlockSpec((1, DM), lambda i: (0, 0)),
            pl.BlockSpec((1, DM), lambda i: (0, 0)),
            pl.BlockSpec((1, DM), lambda i: (0, 0)),
            pl.BlockSpec((S, DM), lambda i: (0, 0)),
        ],
        out_specs=pl.BlockSpec((R, S, DM), lambda i: (i, 0, 0)),
        out_shape=jax.ShapeDtypeStruct((B, S, DM), f32),
    )(pt_sum, es_emb, proc_path_ids, tgt_path_ids,
      numerical, temporal, wcat, wn, wt, bias2,
      ln_gamma[None, :], ln_beta[None, :], pe_s)
    return out


# speedup vs baseline: 1.3586x; 1.0508x over previous
import functools
import math

import jax
import jax.numpy as jnp
from jax import lax
from jax.experimental import pallas as pl
from jax.experimental.pallas import tpu as pltpu
from jax.experimental.pallas import tpu_sc as plsc

_BH = 4
_WT = 32
_R = 16


def _tc_body(pa_ref, tb_ref, pid_ref, tid_ref, num_ref, temp_ref,
             w_ref, wn_ref, wt_ref, bias_ref, g_ref, b_ref, pe_ref, o_ref):
    M = pa_ref.shape[0]
    S = pe_ref.shape[0]
    R = M // S
    P = pid_ref.shape[2]
    D = pa_ref.shape[1] // 2
    pav = pa_ref[...]
    tbv = tb_ref[:, :D]
    pidf = (pid_ref[...].reshape(M, P) != 0).astype(jnp.float32)
    tidf = (tid_ref[...].reshape(M, P) != 0).astype(jnp.float32)
    rp = 1.0 / jnp.maximum(jnp.sum(pidf, axis=1, keepdims=True), 1.0)
    rt = 1.0 / jnp.maximum(jnp.sum(tidf, axis=1, keepdims=True), 1.0)
    lane = lax.broadcasted_iota(jnp.int32, (1, 2 * D), 1)
    scale = jnp.where(lane < D, rp, 1.0)
    cat = jnp.concatenate([pav * scale, tbv * rt], axis=1)
    x = jnp.dot(cat, w_ref[...], preferred_element_type=jnp.float32)
    x = x + jnp.dot(num_ref[...].reshape(M, num_ref.shape[2]), wn_ref[...],
                    preferred_element_type=jnp.float32)
    x = x + jnp.dot(temp_ref[...].reshape(M, temp_ref.shape[2]), wt_ref[...],
                    preferred_element_type=jnp.float32)
    x = x + bias_ref[...]
    mu = jnp.mean(x, axis=1, keepdims=True)
    xc = x - mu
    var = jnp.mean(xc * xc, axis=1, keepdims=True)
    y = xc * lax.rsqrt(var + 1e-5) * g_ref[...] + b_ref[...]
    y = 0.5 * y * (1.0 + lax.erf(y * (1.0 / math.sqrt(2.0))))
    o_ref[...] = y.reshape(R, S, y.shape[1]) + pe_ref[...][None]


def kernel(event_type_ids, proc_path_ids, tgt_path_ids, signing_ids,
           numerical, temporal, event_table, proc_path_table, tgt_path_table,
           signing_table, num_W, num_b, temp_W, temp_b, proj_W, proj_b,
           ln_gamma, ln_beta, pe):
    B, S, P = proc_path_ids.shape
    BT = B * S
    E = event_table.shape[1]
    D = proc_path_table.shape[1]
    SG = signing_table.shape[1]
    DM = proj_W.shape[1]
    f32 = jnp.float32

    info = plsc.get_sparse_core_info()
    NC, NS = info.num_cores, info.num_subcores
    NW = NC * NS
    bw = B // NW
    BH = _BH
    nhg = bw // BH
    HT = BH * S
    WT = _WT
    NWAVE = HT // WT
    NIR = HT * P // 128
    NPAIR = NWAVE // 2

    mesh = plsc.VectorSubcoreMesh(core_axis_name="c", subcore_axis_name="s")

    def make_sc(with_es):

        @functools.partial(
            pl.kernel,
            out_type=jax.ShapeDtypeStruct((BT, 2 * D), f32),
            mesh=mesh,
            scratch_types=[
                pltpu.VMEM((BH, S, P), jnp.int32),
                pltpu.VMEM((NIR, 128), jnp.int32),
                pltpu.VMEM((2, WT * P, D), jnp.bfloat16),
                pltpu.VMEM((2, WT, D), f32),
                pltpu.VMEM((BH, S), jnp.int32),
                pltpu.VMEM((BH, S), jnp.int32),
                pltpu.VMEM((2, S, E), f32),
                pltpu.VMEM((2, S, SG), f32),
                pltpu.SemaphoreType.DMA,
                pltpu.SemaphoreType.DMA,
                pltpu.SemaphoreType.DMA,
                pltpu.SemaphoreType.DMA,
                pltpu.SemaphoreType.DMA,
            ],
            compiler_params=pltpu.CompilerParams(
                use_tc_tiling_on_sc=False, needs_layout_passes=False),
        )
        def sc_call(ids_h, eids_h, sids_h, tab_h, etab_h, stab_h, out,
                    pstg, pidx, pbuf, ptb, eidg, sidg, erow, srow,
                    sem_g0, sem_g1, sem_o, sem_es, sem_eso):
            wid = lax.axis_index("s") * NC + lax.axis_index("c")
            b_base = wid * bw
            lanes = lax.iota(jnp.int32, 16)
            rb = lanes >> 3
            cb = lanes & 7
            gsem = (sem_g0, sem_g1)

            def fire_wave(w, k):
                for u in range(2):
                    hsl = pl.ds(u * 128, 128)
                    pltpu.async_copy(tab_h.at[pidx.at[2 * w + u]],
                                     pbuf.at[k, hsl], gsem[k])

            def wait_wave(k):
                for u in range(2):
                    hsl = pl.ds(u * 128, 128)
                    pltpu.make_async_copy(tab_h.at[pidx.at[0]],
                                          pbuf.at[k, hsl], gsem[k]).wait()

            def compute_wave(k, hgtok, w):
                @pl.loop(0, WT)
                def _tok(t):
                    base = t * P

                    def tree8(sl):
                        r = [pbuf[k, base + q, sl] for q in range(P)]
                        while len(r) > 1:
                            r = [r[2 * q] + r[2 * q + 1]
                                 for q in range(len(r) // 2)]
                        return plsc.unpack(
                            r[0], format=plsc.PackFormat.INTERLEAVED)

                    for c2 in range(D // 32):
                        pa, pb = tree8(pl.ds(c2 * 32, 32))
                        ptb[k, t, pl.ds(c2 * 32, 16)] = pa
                        ptb[k, t, pl.ds(c2 * 32 + 16, 16)] = pb

                pltpu.async_copy(
                    ptb.at[k],
                    out.at[pl.ds(hgtok + w * WT, WT), pl.ds(0, D)], sem_o)

            def drain_out():
                pltpu.make_async_copy(
                    ptb.at[0], out.at[pl.ds(0, WT), pl.ds(0, D)],
                    sem_o).wait()

            @pl.loop(0, nhg)
            def _hg(h):
                b0 = b_base + h * BH
                hgtok = pl.multiple_of(b0 * S, 8)
                kk_b = S * P // 16
                pltpu.sync_copy(ids_h.at[pl.ds(b0, BH), :, :], pstg)

                @pl.loop(0, HT * P // 16)
                def _rp(kk):
                    b_ = kk // kk_b
                    bv = lanes * 0 + b_
                    sv = (kk % kk_b) * 2 + rb
                    r_ = kk >> 3
                    c_ = (kk & 7) * 16
                    pidx[r_, pl.ds(c_, 16)] = plsc.load_gather(
                        pstg, [bv, sv, cb])

                fire_wave(0, 0)

                if with_es:
                    pltpu.sync_copy(eids_h.at[pl.ds(b0, BH), :], eidg)
                    pltpu.sync_copy(sids_h.at[pl.ds(b0, BH), :], sidg)
                    for r in range(BH):
                        k = r % 2
                        if r >= 2:
                            pltpu.make_async_copy(
                                erow.at[k],
                                out.at[pl.ds(0, S), pl.ds(D, E)],
                                sem_eso).wait()
                            pltpu.make_async_copy(
                                srow.at[k],
                                out.at[pl.ds(0, S), pl.ds(D + E, SG)],
                                sem_eso).wait()
                        cps = []
                        for (lo, ln) in ((0, 128), (128, S - 128)):
                            cps.append(pltpu.async_copy(
                                etab_h.at[eidg.at[r, pl.ds(lo, ln)]],
                                erow.at[k, pl.ds(lo, ln)], sem_es))
                            cps.append(pltpu.async_copy(
                                stab_h.at[sidg.at[r, pl.ds(lo, ln)]],
                                srow.at[k, pl.ds(lo, ln)], sem_es))
                        for cp in cps:
                            cp.wait()
                        rtok = pl.multiple_of(hgtok + r * S, 8)
                        pltpu.async_copy(
                            erow.at[k],
                            out.at[pl.ds(rtok, S), pl.ds(D, E)], sem_eso)
                        pltpu.async_copy(
                            srow.at[k],
                            out.at[pl.ds(rtok, S), pl.ds(D + E, SG)],
                            sem_eso)

                @pl.loop(0, NPAIR)
                def _pair(j):
                    w0 = 2 * j
                    fire_wave(w0 + 1, 1)
                    wait_wave(0)

                    @pl.when(j > 0)
                    def _lazy():
                        drain_out()
                        drain_out()

                    compute_wave(0, hgtok, w0)
                    fire_wave(w0 + 2, 0)
                    wait_wave(1)
                    compute_wave(1, hgtok, w0 + 1)

                wait_wave(0)
                drain_out()
                drain_out()
                compute_wave(0, hgtok, NWAVE - 1)
                drain_out()
                if with_es:
                    for _ in range(4):
                        pltpu.make_async_copy(
                            erow.at[0], out.at[pl.ds(0, S), pl.ds(D, E)],
                            sem_eso).wait()

        return sc_call

    pa_out = make_sc(True)(
        proc_path_ids, event_type_ids, signing_ids,
        proc_path_table.astype(jnp.bfloat16), event_table, signing_table)
    tb_out = make_sc(False)(
        tgt_path_ids, event_type_ids, signing_ids,
        tgt_path_table.astype(jnp.bfloat16), event_table, signing_table)

    NW_ = proj_W[E + 2 * D + SG:E + 2 * D + SG + num_W.shape[1]]
    TW_ = proj_W[E + 2 * D + SG + num_W.shape[1]:]
    wn = num_W @ NW_
    wt = temp_W @ TW_
    bias2 = (proj_b + num_b @ NW_ + temp_b @ TW_)[None, :]
    perm = jnp.concatenate(
        [jnp.arange(0, 32, 2, dtype=jnp.int32) + 32 * blk2 + off
         for blk2 in range(D // 32) for off in (0, 1)])
    wp = proj_W[E:E + D][perm]
    wtg = proj_W[E + D:E + 2 * D][perm]
    wcat = jnp.concatenate([wp, proj_W[:E],
                            proj_W[E + 2 * D:E + 2 * D + SG], wtg],
                           axis=0)
    pe_s = pe[0, :S, :]

    R = _R
    M = R * S
    out = pl.pallas_call(
        _tc_body,
        grid=(B // R,),
        in_specs=[
            pl.BlockSpec((M, 2 * D), lambda i: (i, 0)),
            pl.BlockSpec((M, 2 * D), lambda i: (i, 0)),
            pl.BlockSpec((R, S, P), lambda i: (i, 0, 0)),
            pl.BlockSpec((R, S, P), lambda i: (i, 0, 0)),
            pl.BlockSpec((R, S, numerical.shape[2]), lambda i: (i, 0, 0)),
            pl.BlockSpec((R, S, temporal.shape[2]), lambda i: (i, 0, 0)),
            pl.BlockSpec((E + 2 * D + SG, DM), lambda i: (0, 0)),
            pl.BlockSpec((numerical.shape[2], DM), lambda i: (0, 0)),
            pl.BlockSpec((temporal.shape[2], DM), lambda i: (0, 0)),
            pl.BlockSpec((1, DM), lambda i: (0, 0)),
            pl.BlockSpec((1, DM), lambda i: (0, 0)),
            pl.BlockSpec((1, DM), lambda i: (0, 0)),
            pl.BlockSpec((S, DM), lambda i: (0, 0)),
        ],
        out_specs=pl.BlockSpec((R, S, DM), lambda i: (i, 0, 0)),
        out_shape=jax.ShapeDtypeStruct((B, S, DM), f32),
    )(pa_out, tb_out, proc_path_ids, tgt_path_ids,
      numerical, temporal, wcat, wn, wt, bias2,
      ln_gamma[None, :], ln_beta[None, :], pe_s)
    return out


# speedup vs baseline: 1.3622x; 1.0027x over previous
import functools
import math

import jax
import jax.numpy as jnp
from jax import lax
from jax.experimental import pallas as pl
from jax.experimental.pallas import tpu as pltpu
from jax.experimental.pallas import tpu_sc as plsc

_BH = 4
_WT = 32
_R = 16


def _tc_body(pa_ref, tb_ref, pid_ref, tid_ref, num_ref, temp_ref,
             w_ref, wn_ref, wt_ref, bias_ref, g_ref, b_ref, pe_ref, o_ref):
    M = pa_ref.shape[0]
    S = pe_ref.shape[0]
    R = M // S
    P = pid_ref.shape[2]
    D = pa_ref.shape[1] // 2
    pav = pa_ref[...]
    tbv = tb_ref[:, :D]
    pidf = (pid_ref[...].reshape(M, P) != 0).astype(jnp.float32)
    tidf = (tid_ref[...].reshape(M, P) != 0).astype(jnp.float32)
    rp = 1.0 / jnp.maximum(jnp.sum(pidf, axis=1, keepdims=True), 1.0)
    rt = 1.0 / jnp.maximum(jnp.sum(tidf, axis=1, keepdims=True), 1.0)
    lane = lax.broadcasted_iota(jnp.int32, (1, 2 * D), 1)
    scale = jnp.where(lane < D, rp, 1.0)
    cat = jnp.concatenate([pav * scale, tbv * rt], axis=1)
    x = jnp.dot(cat, w_ref[...], preferred_element_type=jnp.float32)
    x = x + jnp.dot(num_ref[...].reshape(M, num_ref.shape[2]), wn_ref[...],
                    preferred_element_type=jnp.float32)
    x = x + jnp.dot(temp_ref[...].reshape(M, temp_ref.shape[2]), wt_ref[...],
                    preferred_element_type=jnp.float32)
    x = x + bias_ref[...]
    mu = jnp.mean(x, axis=1, keepdims=True)
    xc = x - mu
    var = jnp.mean(xc * xc, axis=1, keepdims=True)
    y = xc * lax.rsqrt(var + 1e-5) * g_ref[...] + b_ref[...]
    y = 0.5 * y * (1.0 + lax.erf(y * (1.0 / math.sqrt(2.0))))
    o_ref[...] = y.reshape(R, S, y.shape[1]) + pe_ref[...][None]


def kernel(event_type_ids, proc_path_ids, tgt_path_ids, signing_ids,
           numerical, temporal, event_table, proc_path_table, tgt_path_table,
           signing_table, num_W, num_b, temp_W, temp_b, proj_W, proj_b,
           ln_gamma, ln_beta, pe):
    B, S, P = proc_path_ids.shape
    BT = B * S
    E = event_table.shape[1]
    D = proc_path_table.shape[1]
    SG = signing_table.shape[1]
    DM = proj_W.shape[1]
    f32 = jnp.float32

    info = plsc.get_sparse_core_info()
    NC, NS = info.num_cores, info.num_subcores
    NW = NC * NS
    bw = B // NW
    BH = _BH
    nhg = bw // BH
    HT = BH * S
    WT = _WT
    NWAVE = HT // WT
    NIR = HT * P // 128
    NPAIR = NWAVE // 2

    mesh = plsc.VectorSubcoreMesh(core_axis_name="c", subcore_axis_name="s")

    def make_sc(with_es):

        @functools.partial(
            pl.kernel,
            out_type=jax.ShapeDtypeStruct((BT, 2 * D), f32),
            mesh=mesh,
            scratch_types=[
                pltpu.VMEM((BH, S, P), jnp.int32),
                pltpu.VMEM((NIR, 128), jnp.int32),
                pltpu.VMEM((2, WT * P, D), jnp.bfloat16),
                pltpu.VMEM((2, WT, D), f32),
                pltpu.VMEM((BH, S), jnp.int32),
                pltpu.VMEM((BH, S), jnp.int32),
                pltpu.VMEM((2, S, E), f32),
                pltpu.VMEM((2, S, SG), f32),
                pltpu.SemaphoreType.DMA,
                pltpu.SemaphoreType.DMA,
                pltpu.SemaphoreType.DMA,
                pltpu.SemaphoreType.DMA,
                pltpu.SemaphoreType.DMA,
            ],
            compiler_params=pltpu.CompilerParams(
                use_tc_tiling_on_sc=False, needs_layout_passes=False),
        )
        def sc_call(ids_h, eids_h, sids_h, tab_h, etab_h, stab_h, out,
                    pstg, pidx, pbuf, ptb, eidg, sidg, erow, srow,
                    sem_g0, sem_g1, sem_o, sem_es, sem_eso):
            wid = lax.axis_index("s") * NC + lax.axis_index("c")
            b_base = wid * bw
            lanes = lax.iota(jnp.int32, 16)
            rb = lanes >> 3
            cb = lanes & 7
            gsem = (sem_g0, sem_g1)

            def fire_wave(w, k):
                for u in range(2):
                    hsl = pl.ds(u * 128, 128)
                    pltpu.async_copy(tab_h.at[pidx.at[2 * w + u]],
                                     pbuf.at[k, hsl], gsem[k])

            def wait_wave(k):
                for u in range(2):
                    hsl = pl.ds(u * 128, 128)
                    pltpu.make_async_copy(tab_h.at[pidx.at[0]],
                                          pbuf.at[k, hsl], gsem[k]).wait()

            def compute_wave(k, hgtok, w):
                @pl.loop(0, WT, unroll=4)
                def _tok(t):
                    base = t * P

                    def tree8(sl):
                        r = [pbuf[k, base + q, sl] for q in range(P)]
                        while len(r) > 1:
                            r = [r[2 * q] + r[2 * q + 1]
                                 for q in range(len(r) // 2)]
                        return plsc.unpack(
                            r[0], format=plsc.PackFormat.INTERLEAVED)

                    for c2 in range(D // 32):
                        pa, pb = tree8(pl.ds(c2 * 32, 32))
                        ptb[k, t, pl.ds(c2 * 32, 16)] = pa
                        ptb[k, t, pl.ds(c2 * 32 + 16, 16)] = pb

                pltpu.async_copy(
                    ptb.at[k],
                    out.at[pl.ds(hgtok + w * WT, WT), pl.ds(0, D)], sem_o)

            def drain_out():
                pltpu.make_async_copy(
                    ptb.at[0], out.at[pl.ds(0, WT), pl.ds(0, D)],
                    sem_o).wait()

            @pl.loop(0, nhg)
            def _hg(h):
                b0 = b_base + h * BH
                hgtok = pl.multiple_of(b0 * S, 8)
                kk_b = S * P // 16
                pltpu.sync_copy(ids_h.at[pl.ds(b0, BH), :, :], pstg)

                @pl.loop(0, HT * P // 16, unroll=8)
                def _rp(kk):
                    b_ = kk // kk_b
                    bv = lanes * 0 + b_
                    sv = (kk % kk_b) * 2 + rb
                    r_ = kk >> 3
                    c_ = (kk & 7) * 16
                    pidx[r_, pl.ds(c_, 16)] = plsc.load_gather(
                        pstg, [bv, sv, cb])

                fire_wave(0, 0)

                if with_es:
                    pltpu.sync_copy(eids_h.at[pl.ds(b0, BH), :], eidg)
                    pltpu.sync_copy(sids_h.at[pl.ds(b0, BH), :], sidg)
                    for r in range(BH):
                        k = r % 2
                        if r >= 2:
                            pltpu.make_async_copy(
                                erow.at[k],
                                out.at[pl.ds(0, S), pl.ds(D, E)],
                                sem_eso).wait()
                            pltpu.make_async_copy(
                                srow.at[k],
                                out.at[pl.ds(0, S), pl.ds(D + E, SG)],
                                sem_eso).wait()
                        cps = []
                        for (lo, ln) in ((0, 128), (128, S - 128)):
                            cps.append(pltpu.async_copy(
                                etab_h.at[eidg.at[r, pl.ds(lo, ln)]],
                                erow.at[k, pl.ds(lo, ln)], sem_es))
                            cps.append(pltpu.async_copy(
                                stab_h.at[sidg.at[r, pl.ds(lo, ln)]],
                                srow.at[k, pl.ds(lo, ln)], sem_es))
                        for cp in cps:
                            cp.wait()
                        rtok = pl.multiple_of(hgtok + r * S, 8)
                        pltpu.async_copy(
                            erow.at[k],
                            out.at[pl.ds(rtok, S), pl.ds(D, E)], sem_eso)
                        pltpu.async_copy(
                            srow.at[k],
                            out.at[pl.ds(rtok, S), pl.ds(D + E, SG)],
                            sem_eso)

                @pl.loop(0, NPAIR)
                def _pair(j):
                    w0 = 2 * j
                    fire_wave(w0 + 1, 1)
                    wait_wave(0)

                    @pl.when(j > 0)
                    def _lazy():
                        drain_out()
                        drain_out()

                    compute_wave(0, hgtok, w0)
                    fire_wave(w0 + 2, 0)
                    wait_wave(1)
                    compute_wave(1, hgtok, w0 + 1)

                wait_wave(0)
                drain_out()
                drain_out()
                compute_wave(0, hgtok, NWAVE - 1)
                drain_out()
                if with_es:
                    for _ in range(4):
                        pltpu.make_async_copy(
                            erow.at[0], out.at[pl.ds(0, S), pl.ds(D, E)],
                            sem_eso).wait()

        return sc_call

    pa_out = make_sc(True)(
        proc_path_ids, event_type_ids, signing_ids,
        proc_path_table.astype(jnp.bfloat16), event_table, signing_table)
    tb_out = make_sc(False)(
        tgt_path_ids, event_type_ids, signing_ids,
        tgt_path_table.astype(jnp.bfloat16), event_table, signing_table)

    NW_ = proj_W[E + 2 * D + SG:E + 2 * D + SG + num_W.shape[1]]
    TW_ = proj_W[E + 2 * D + SG + num_W.shape[1]:]
    wn = num_W @ NW_
    wt = temp_W @ TW_
    bias2 = (proj_b + num_b @ NW_ + temp_b @ TW_)[None, :]
    perm = jnp.concatenate(
        [jnp.arange(0, 32, 2, dtype=jnp.int32) + 32 * blk2 + off
         for blk2 in range(D // 32) for off in (0, 1)])
    wp = proj_W[E:E + D][perm]
    wtg = proj_W[E + D:E + 2 * D][perm]
    wcat = jnp.concatenate([wp, proj_W[:E],
                            proj_W[E + 2 * D:E + 2 * D + SG], wtg],
                           axis=0)
    pe_s = pe[0, :S, :]

    R = _R
    M = R * S
    out = pl.pallas_call(
        _tc_body,
        grid=(B // R,),
        in_specs=[
            pl.BlockSpec((M, 2 * D), lambda i: (i, 0)),
            pl.BlockSpec((M, 2 * D), lambda i: (i, 0)),
            pl.BlockSpec((R, S, P), lambda i: (i, 0, 0)),
            pl.BlockSpec((R, S, P), lambda i: (i, 0, 0)),
            pl.BlockSpec((R, S, numerical.shape[2]), lambda i: (i, 0, 0)),
            pl.BlockSpec((R, S, temporal.shape[2]), lambda i: (i, 0, 0)),
            pl.BlockSpec((E + 2 * D + SG, DM), lambda i: (0, 0)),
            pl.BlockSpec((numerical.shape[2], DM), lambda i: (0, 0)),
            pl.BlockSpec((temporal.shape[2], DM), lambda i: (0, 0)),
            pl.BlockSpec((1, DM), lambda i: (0, 0)),
            pl.BlockSpec((1, DM), lambda i: (0, 0)),
            pl.BlockSpec((1, DM), lambda i: (0, 0)),
            pl.BlockSpec((S, DM), lambda i: (0, 0)),
        ],
        out_specs=pl.BlockSpec((R, S, DM), lambda i: (i, 0, 0)),
        out_shape=jax.ShapeDtypeStruct((B, S, DM), f32),
    )(pa_out, tb_out, proc_path_ids, tgt_path_ids,
      numerical, temporal, wcat, wn, wt, bias2,
      ln_gamma[None, :], ln_beta[None, :], pe_s)
    return out
